# baseline (speedup 1.0000x reference)
import functools

import jax
import jax.numpy as jnp
from jax import lax
from jax.experimental import pallas as pl
from jax.experimental.pallas import tpu as pltpu
from jax.experimental.pallas import tpu_sc as plsc

DIM = 64
B = 16384
LANES = 16
NROWS = 500000


@functools.lru_cache(maxsize=None)
def _build_kernel():
    info = plsc.get_sparse_core_info()
    nc, ns = info.num_cores, info.num_subcores
    nw = nc * ns
    b_per_w = B // nw
    chunk = b_per_w // 2
    n_grp = chunk // 128
    mesh = plsc.VectorSubcoreMesh(core_axis_name="c", subcore_axis_name="s")

    @functools.partial(
        pl.kernel,
        mesh=mesh,
        out_type=(
            jax.ShapeDtypeStruct((B, DIM), jnp.float32),
            jax.ShapeDtypeStruct((B, DIM), jnp.float32),
        ),
        scratch_types=[
            pltpu.VMEM((b_per_w,), jnp.int32),
            pltpu.VMEM((b_per_w,), jnp.int32),
            pltpu.VMEM((n_grp, 128), jnp.int32),
            pltpu.VMEM((n_grp, 128), jnp.int32),
            pltpu.VMEM((chunk, 2 * DIM), jnp.float32),
            pltpu.VMEM((chunk, 2 * DIM), jnp.float32),
            pltpu.VMEM((chunk, DIM), jnp.float32),
            pltpu.SemaphoreType.DMA,
            pltpu.SemaphoreType.DMA,
        ],
    )
    def gather_kernel(user_hbm, item_hbm, ut_hbm, it_hbm, out_u, out_i,
                      idx_u, idx_i, pidx_u, pidx_i, pairs_u, pairs_i,
                      st64, sem_u, sem_i):
        wid = lax.axis_index("s") * nc + lax.axis_index("c")
        base = wid * b_per_w
        pltpu.sync_copy(user_hbm.at[pl.ds(base, b_per_w)], idx_u)
        pltpu.sync_copy(item_hbm.at[pl.ds(base, b_per_w)], idx_i)

        def fire_chunk(tbl_hbm, idx_v, pidx_v, pairs_v, sem, c0):
            def mkpidx(w, _):
                j0 = w * LANES
                v = idx_v[pl.ds(c0 + j0, LANES)] >> 1
                pidx_v[w // (128 // LANES), pl.ds((j0 % 128), LANES)] = v
                return ()

            lax.fori_loop(0, chunk // LANES, mkpidx, (), unroll=False)
            for g in range(n_grp):
                pltpu.async_copy(
                    tbl_hbm.at[pidx_v.at[g]],
                    pairs_v.at[pl.ds(g * 128, 128)],
                    sem,
                )

        def drain_chunk(tbl_hbm, pairs_v, sem):
            pltpu.make_async_copy(
                tbl_hbm.at[pl.ds(0, chunk)], pairs_v, sem
            ).wait()

        def select_write(idx_v, pairs_v, out, c0):
            def wave(w, _):
                j0 = w * LANES
                offv = (idx_v[pl.ds(c0 + j0, LANES)] & 1) * DIM
                for k in range(LANES):
                    off = offv[k]
                    for m in range(DIM // LANES):
                        st64[j0 + k, pl.ds(m * LANES, LANES)] = (
                            pairs_v[j0 + k, pl.ds(off + m * LANES, LANES)]
                        )
                return ()

            lax.fori_loop(0, chunk // LANES, wave, (), unroll=False)
            pltpu.sync_copy(st64, out.at[pl.ds(base + c0, chunk)])

        fire_chunk(ut_hbm, idx_u, pidx_u, pairs_u, sem_u, 0)
        fire_chunk(it_hbm, idx_i, pidx_i, pairs_i, sem_i, 0)
        drain_chunk(ut_hbm, pairs_u, sem_u)
        select_write(idx_u, pairs_u, out_u, 0)
        fire_chunk(ut_hbm, idx_u, pidx_u, pairs_u, sem_u, chunk)
        drain_chunk(it_hbm, pairs_i, sem_i)
        select_write(idx_i, pairs_i, out_i, 0)
        fire_chunk(it_hbm, idx_i, pidx_i, pairs_i, sem_i, chunk)
        drain_chunk(ut_hbm, pairs_u, sem_u)
        select_write(idx_u, pairs_u, out_u, chunk)
        drain_chunk(it_hbm, pairs_i, sem_i)
        select_write(idx_i, pairs_i, out_i, chunk)

    return gather_kernel


def kernel(user, item, user_table, item_table):
    ut2 = user_table.reshape(NROWS, 2 * DIM)
    it2 = item_table.reshape(NROWS, 2 * DIM)
    return _build_kernel()(user, item, ut2, it2)


# speedup vs baseline: 1.3628x; 1.3628x over previous
import functools

import jax
import jax.numpy as jnp
from jax import lax
from jax.experimental import pallas as pl
from jax.experimental.pallas import tpu as pltpu

DIM = 64
B = 16384
CH = 2048
NCH = B // CH


def _gather_kernel(user_smem, item_smem, ut_hbm, it_hbm, out_u, out_i,
                   buf_a, buf_b, sem_g, sem_wa, sem_wb):
    bufs = (buf_a, buf_b)
    wsems = (sem_wa, sem_wb)
    plan = [(user_smem, ut_hbm, out_u), (item_smem, it_hbm, out_i)]
    pending = {}

    g = 0
    for idx_smem, tbl, out in plan:
        for c in range(NCH):
            par = g % 2
            buf, wsem = bufs[par], wsems[par]
            if par in pending:
                pbuf, pout, pbase = pending.pop(par)
                pltpu.make_async_copy(
                    pbuf, pout.at[pl.ds(pbase, CH)], wsem
                ).wait()

            def fire(j, _, idx_smem=idx_smem, tbl=tbl, buf=buf, c=c):
                i = idx_smem[c * CH + j]
                pltpu.make_async_copy(
                    tbl.at[pl.ds(i, 1)], buf.at[pl.ds(j, 1)], sem_g
                ).start()
                return ()

            lax.fori_loop(0, CH, fire, (), unroll=8)

            def drain(j, _, tbl=tbl, buf=buf):
                pltpu.make_async_copy(
                    tbl.at[pl.ds(0, 1)], buf.at[pl.ds(j, 1)], sem_g
                ).wait()
                return ()

            lax.fori_loop(0, CH, drain, (), unroll=8)

            pltpu.make_async_copy(
                buf, out.at[pl.ds(c * CH, CH)], wsem
            ).start()
            pending[par] = (buf, out, c * CH)
            g += 1

    for par, (pbuf, pout, pbase) in pending.items():
        pltpu.make_async_copy(
            pbuf, pout.at[pl.ds(pbase, CH)], wsems[par]
        ).wait()


@functools.lru_cache(maxsize=None)
def _build_kernel():
    return pl.pallas_call(
        _gather_kernel,
        out_shape=(
            jax.ShapeDtypeStruct((B, DIM), jnp.float32),
            jax.ShapeDtypeStruct((B, DIM), jnp.float32),
        ),
        in_specs=[
            pl.BlockSpec(memory_space=pltpu.SMEM),
            pl.BlockSpec(memory_space=pltpu.SMEM),
            pl.BlockSpec(memory_space=pl.ANY),
            pl.BlockSpec(memory_space=pl.ANY),
        ],
        out_specs=(
            pl.BlockSpec(memory_space=pl.ANY),
            pl.BlockSpec(memory_space=pl.ANY),
        ),
        scratch_shapes=[
            pltpu.VMEM((CH, DIM), jnp.float32),
            pltpu.VMEM((CH, DIM), jnp.float32),
            pltpu.SemaphoreType.DMA,
            pltpu.SemaphoreType.DMA,
            pltpu.SemaphoreType.DMA,
        ],
    )


def kernel(user, item, user_table, item_table):
    return _build_kernel()(user, item, user_table, item_table)


# speedup vs baseline: 1.3721x; 1.0069x over previous
import functools

import jax
import jax.numpy as jnp
from jax import lax
from jax.experimental import pallas as pl
from jax.experimental.pallas import tpu as pltpu

DIM = 64
B = 16384
CH = 4096
NCH = B // CH


def _gather_kernel(user_smem, item_smem, ut_hbm, it_hbm, out_u, out_i,
                   buf_a, buf_b, sem_g, sem_wa, sem_wb):
    bufs = (buf_a, buf_b)
    wsems = (sem_wa, sem_wb)
    plan = [(user_smem, ut_hbm, out_u), (item_smem, it_hbm, out_i)]
    pending = {}

    g = 0
    for idx_smem, tbl, out in plan:
        for c in range(NCH):
            par = g % 2
            buf, wsem = bufs[par], wsems[par]
            if par in pending:
                pbuf, pout, pbase = pending.pop(par)
                pltpu.make_async_copy(
                    pbuf, pout.at[pl.ds(pbase, CH)], wsem
                ).wait()

            def fire(j, _, idx_smem=idx_smem, tbl=tbl, buf=buf, c=c):
                i = idx_smem[c * CH + j]
                pltpu.make_async_copy(
                    tbl.at[pl.ds(i, 1)], buf.at[pl.ds(j, 1)], sem_g
                ).start()
                return ()

            lax.fori_loop(0, CH, fire, (), unroll=16)

            pltpu.make_async_copy(
                tbl.at[pl.ds(0, CH)], buf, sem_g
            ).wait()

            pltpu.make_async_copy(
                buf, out.at[pl.ds(c * CH, CH)], wsem
            ).start()
            pending[par] = (buf, out, c * CH)
            g += 1

    for par, (pbuf, pout, pbase) in pending.items():
        pltpu.make_async_copy(
            pbuf, pout.at[pl.ds(pbase, CH)], wsems[par]
        ).wait()


@functools.lru_cache(maxsize=None)
def _build_kernel():
    return pl.pallas_call(
        _gather_kernel,
        out_shape=(
            jax.ShapeDtypeStruct((B, DIM), jnp.float32),
            jax.ShapeDtypeStruct((B, DIM), jnp.float32),
        ),
        in_specs=[
            pl.BlockSpec(memory_space=pltpu.SMEM),
            pl.BlockSpec(memory_space=pltpu.SMEM),
            pl.BlockSpec(memory_space=pl.ANY),
            pl.BlockSpec(memory_space=pl.ANY),
        ],
        out_specs=(
            pl.BlockSpec(memory_space=pl.ANY),
            pl.BlockSpec(memory_space=pl.ANY),
        ),
        scratch_shapes=[
            pltpu.VMEM((CH, DIM), jnp.float32),
            pltpu.VMEM((CH, DIM), jnp.float32),
            pltpu.SemaphoreType.DMA,
            pltpu.SemaphoreType.DMA,
            pltpu.SemaphoreType.DMA,
        ],
    )


def kernel(user, item, user_table, item_table):
    return _build_kernel()(user, item, user_table, item_table)
